# Initial kernel scaffold; baseline (speedup 1.0000x reference)
import jax
import jax.numpy as jnp
from jax.experimental import pallas as pl


def kernel(input_dp, table, ln_gamma, ln_beta):
    raise NotImplementedError("write your pallas kernel here")


# speedup vs baseline: 2.8372x; 2.8372x over previous
import functools
import math

import jax
import jax.numpy as jnp
from jax import lax
from jax.experimental import pallas as pl
from jax.experimental.pallas import tpu as pltpu
from jax.experimental.pallas import tpu_sc as plsc


def _pe_table(L, D):
    def body(o_ref):
        pos = lax.broadcasted_iota(jnp.int32, (L, D), 0).astype(jnp.float32)
        k = lax.broadcasted_iota(jnp.int32, (L, D), 1)
        half = (k // 2).astype(jnp.float32)
        ang = pos * jnp.exp(half * (-2.0 * math.log(10000.0) / D))
        o_ref[...] = jnp.where(k % 2 == 0, jnp.sin(ang), jnp.cos(ang))

    return pl.pallas_call(
        body, out_shape=jax.ShapeDtypeStruct((L, D), jnp.float32)
    )()


def _sc_gather(idx3, table):
    NW, n_ch, CH = idx3.shape
    V, D = table.shape
    SUP = 2
    ROWS = SUP * CH
    NBUF = 4
    n_sup = n_ch // SUP
    per_w = n_ch * CH
    info = plsc.get_sparse_core_info()
    NC = info.num_cores
    mesh = plsc.VectorSubcoreMesh(core_axis_name="c", subcore_axis_name="s")

    @functools.partial(
        pl.kernel,
        out_type=jax.ShapeDtypeStruct((NW * per_w, D), jnp.float32),
        mesh=mesh,
        compiler_params=pltpu.CompilerParams(use_tc_tiling_on_sc=False),
        scratch_types=[
            pltpu.VMEM((n_ch, CH), jnp.int32),
            pltpu.VMEM((ROWS, D), jnp.float32),
            pltpu.VMEM((ROWS, D), jnp.float32),
            pltpu.VMEM((ROWS, D), jnp.float32),
            pltpu.VMEM((ROWS, D), jnp.float32),
            pltpu.SemaphoreType.DMA,
            pltpu.SemaphoreType.DMA,
            pltpu.SemaphoreType.DMA,
            pltpu.SemaphoreType.DMA,
            pltpu.SemaphoreType.DMA,
            pltpu.SemaphoreType.DMA,
            pltpu.SemaphoreType.DMA,
            pltpu.SemaphoreType.DMA,
        ],
    )
    def k(idx_hbm, table_hbm, out_hbm, idx_v,
          r0, r1, r2, r3, g0, g1, g2, g3, s0, s1, s2, s3):
        bufs = [r0, r1, r2, r3]
        gs = [g0, g1, g2, g3]
        ss = [s0, s1, s2, s3]
        wid = lax.axis_index("s") * NC + lax.axis_index("c")
        base = wid * per_w
        pltpu.sync_copy(idx_hbm.at[wid], idx_v)

        def fire_gather(t, b):
            for j in range(SUP):
                pltpu.async_copy(
                    table_hbm.at[idx_v.at[t * SUP + j]],
                    bufs[b].at[pl.ds(j * CH, CH)],
                    gs[b],
                )

        def wait_gather(b):
            for j in range(SUP):
                pltpu.make_async_copy(
                    table_hbm.at[idx_v.at[0]],
                    bufs[b].at[pl.ds(j * CH, CH)],
                    gs[b],
                ).wait()

        def fire_store(t, b):
            pltpu.async_copy(
                bufs[b], out_hbm.at[pl.ds(base + t * ROWS, ROWS)], ss[b]
            )

        def wait_store(b):
            pltpu.make_async_copy(
                bufs[b], out_hbm.at[pl.ds(base, ROWS)], ss[b]
            ).wait()

        fire_gather(0, 0)
        fire_gather(1, 1)

        def outer(ti, _):
            for h in range(NBUF):
                t = ti * NBUF + h
                wait_gather(h)
                fire_store(t, h)
                b2 = (h + 2) % NBUF

                @pl.when(t >= 2)
                def _():
                    wait_store(b2)

                @pl.when(t + 2 < n_sup)
                def _():
                    fire_gather(t + 2, b2)
            return ()

        lax.fori_loop(0, n_sup // NBUF, outer, ())
        wait_store((n_sup - 2) % NBUF)
        wait_store((n_sup - 1) % NBUF)

    return k(idx3, table)


def _ln_tc(x, pe, gamma, beta):
    N, D = x.shape
    L = pe.shape[0]
    R = 3200
    reps = R // L

    def body(x_ref, pe_ref, g_ref, b_ref, o_ref):
        e = x_ref[...].reshape(reps, L, D) + pe_ref[...][None]
        m = jnp.mean(e, axis=-1, keepdims=True)
        c = e - m
        v = jnp.mean(c * c, axis=-1, keepdims=True)
        o = c * lax.rsqrt(v + 1e-5) * g_ref[...] + b_ref[...]
        o_ref[...] = o.reshape(R, D)

    return pl.pallas_call(
        body,
        grid=(N // R,),
        in_specs=[
            pl.BlockSpec((R, D), lambda i: (i, 0)),
            pl.BlockSpec((L, D), lambda i: (0, 0)),
            pl.BlockSpec((1, D), lambda i: (0, 0)),
            pl.BlockSpec((1, D), lambda i: (0, 0)),
        ],
        out_specs=pl.BlockSpec((R, D), lambda i: (i, 0)),
        out_shape=jax.ShapeDtypeStruct((N, D), jnp.float32),
    )(x, pe, gamma, beta)


def kernel(input_dp, table, ln_gamma, ln_beta):
    B, L = input_dp.shape
    V, D = table.shape
    NW, CH = 32, 128
    n_ch = (B * L) // (NW * CH)
    idx3 = input_dp.reshape(NW, n_ch, CH).astype(jnp.int32)
    pe = _pe_table(L, D)
    rows = _sc_gather(idx3, table)
    out = _ln_tc(rows, pe, ln_gamma.reshape(1, D), ln_beta.reshape(1, D))
    return out.reshape(B, L, D)
